# baseline (speedup 1.0000x reference)
import functools

import jax
import jax.numpy as jnp
from jax import lax
from jax.experimental import pallas as pl
from jax.experimental.pallas import tpu as pltpu
from jax.experimental.pallas import tpu_sc as plsc

_VOCAB = 56
_D = 64
_ROWS, _COLS = 16384, 200
_B = _ROWS * _COLS
_NPAIR = _B // 2
_PT_ROWS = _VOCAB * _VOCAB
_NC, _NS = 2, 16
_NW = _NC * _NS
_PT_PER_S = _PT_ROWS // _NS
_SUB = 128
_NBUF = 4
_PAIRS_PER_W = _NPAIR // _NW
_IDXR_PER_W = _PAIRS_PER_W // _SUB

_mesh = plsc.VectorSubcoreMesh(
    core_axis_name="c", subcore_axis_name="s",
    num_cores=_NC, num_subcores=_NS,
)


def _pairidx_body(even_ref, odd_ref, out_ref):
    out_ref[...] = even_ref[...] * _VOCAB + odd_ref[...]


_pairidx = pl.pallas_call(
    _pairidx_body,
    out_shape=jax.ShapeDtypeStruct((_NPAIR // _SUB, _SUB), jnp.int32),
    grid=(8,),
    in_specs=[
        pl.BlockSpec((_NPAIR // _SUB // 8, _SUB), lambda i: (i, 0)),
        pl.BlockSpec((_NPAIR // _SUB // 8, _SUB), lambda i: (i, 0)),
    ],
    out_specs=pl.BlockSpec((_NPAIR // _SUB // 8, _SUB), lambda i: (i, 0)),
)


@functools.partial(
    pl.kernel,
    out_type=jax.ShapeDtypeStruct((_NPAIR, 2 * _D), jnp.float32),
    mesh=_mesh,
    scratch_types=[
        pltpu.VMEM((_VOCAB * _D,), jnp.float32),
        pltpu.VMEM((_IDXR_PER_W // 2, _SUB), jnp.int32),
        pltpu.VMEM((_NBUF * _SUB, 2 * _D), jnp.float32),
        pltpu.VMEM_SHARED((_PT_ROWS, 2 * _D), jnp.float32),
        pltpu.SemaphoreType.DMA,
        pltpu.SemaphoreType.DMA,
        pltpu.SemaphoreType.DMA,
        pltpu.SemaphoreType.DMA,
        pltpu.SemaphoreType.DMA,
        pltpu.SemaphoreType.DMA,
    ],
)
def _embed(pidx_hbm, wflat_hbm, out_hbm,
           wv, idxall, rows_v, pt_sh, isem, gsem, osem0, osem1, osem2, osem3):
    cid = lax.axis_index("c")
    sid = lax.axis_index("s")
    wid = sid * _NC + cid
    idxr0 = wid * _IDXR_PER_W
    pair0 = wid * _PAIRS_PER_W
    osems = (osem0, osem1, osem2, osem3)

    _HALF = _IDXR_PER_W // 2
    idesc = pltpu.async_copy(
        pidx_hbm.at[pl.ds(idxr0, _HALF)], idxall, isem)

    pltpu.sync_copy(wflat_hbm, wv)
    p0 = sid * _PT_PER_S

    def build_row(r, carry):
        a = (p0 + r) // _VOCAB
        b = (p0 + r) % _VOCAB
        for q in range(4):
            rows_v[r, pl.ds(q * 16, 16)] = wv[pl.ds(a * _D + q * 16, 16)]
            rows_v[r, pl.ds(_D + q * 16, 16)] = wv[pl.ds(b * _D + q * 16, 16)]
        return carry

    lax.fori_loop(0, _PT_PER_S, build_row, 0)
    pltpu.sync_copy(rows_v.at[pl.ds(0, _PT_PER_S)], pt_sh.at[pl.ds(p0, _PT_PER_S)])
    idesc.wait()
    plsc.subcore_barrier()

    def fire_gather(r, b):
        pltpu.async_copy(
            pt_sh.at[idxall.at[r]],
            rows_v.at[pl.ds(b * _SUB, _SUB)],
            gsem,
        )

    def fire_write(g, b):
        pltpu.async_copy(
            rows_v.at[pl.ds(b * _SUB, _SUB)],
            out_hbm.at[pl.ds(pair0 + g * _SUB, _SUB)],
            osems[b],
        )

    def drain_write(b):
        pltpu.make_async_copy(
            rows_v.at[pl.ds(b * _SUB, _SUB)],
            out_hbm.at[pl.ds(0, _SUB)],
            osems[b],
        ).wait()

    def drain_gather(r, b):
        pltpu.make_async_copy(
            pt_sh.at[idxall.at[r]],
            rows_v.at[pl.ds(b * _SUB, _SUB)],
            gsem,
        ).wait()

    for block in range(2):
        g0 = block * _HALF
        if block > 0:
            pltpu.sync_copy(
                pidx_hbm.at[pl.ds(idxr0 + g0, _HALF)], idxall)

        def body(h, carry, g0=g0):
            for b in range(_NBUF):
                r = _NBUF * h + b

                @pl.when(h >= 1)
                def _():
                    drain_write(b)

                fire_gather(r, b)
                pb = (b - 1) % _NBUF
                if b == 0:
                    @pl.when(h >= 1)
                    def _():
                        drain_gather(r - 1, pb)
                        fire_write(g0 + r - 1, pb)
                else:
                    drain_gather(r - 1, pb)
                    fire_write(g0 + r - 1, pb)
            return carry

        lax.fori_loop(0, _HALF // _NBUF, body, 0)
        last_b = _NBUF - 1
        drain_gather(_HALF - 1, last_b)
        fire_write(g0 + _HALF - 1, last_b)
        for b in range(_NBUF):
            drain_write(b)


def _transpose_body(in_ref, out_ref):
    blk = in_ref[...].reshape(_SUB, 2 * _D)
    out_ref[...] = blk.T.reshape(2, _D, _SUB)


_transpose = pl.pallas_call(
    _transpose_body,
    out_shape=jax.ShapeDtypeStruct((_COLS, _D, _ROWS), jnp.float32),
    grid=(_COLS // 2, _ROWS // _SUB),
    in_specs=[
        pl.BlockSpec((1, _SUB, 2 * _D), lambda p, c: (p, c, 0)),
    ],
    out_specs=pl.BlockSpec((2, _D, _SUB), lambda p, c: (p, 0, c)),
)


def kernel(x, weight):
    x3 = x.reshape(_ROWS, _COLS // 2, 2).astype(jnp.int32)
    even = x3[:, :, 0].T.reshape(_NPAIR // _SUB, _SUB)
    odd = x3[:, :, 1].T.reshape(_NPAIR // _SUB, _SUB)
    pidx = _pairidx(even, odd)
    wflat = weight.reshape(_VOCAB * _D)
    pairs = _embed(pidx, wflat)
    out3 = _transpose(pairs.reshape(_COLS // 2, _ROWS, 2 * _D))
    return jnp.transpose(out3, (2, 0, 1))


# speedup vs baseline: 5.6933x; 5.6933x over previous
import functools

import jax
import jax.numpy as jnp
from jax import lax
from jax.experimental import pallas as pl
from jax.experimental.pallas import tpu as pltpu
from jax.experimental.pallas import tpu_sc as plsc

_VOCAB = 56
_D = 64
_ROWS, _COLS = 16384, 200
_B = _ROWS * _COLS
_NPAIR = _B // 2
_PT_ROWS = _VOCAB * _VOCAB
_NC, _NS = 2, 16
_NW = _NC * _NS
_PT_PER_S = _PT_ROWS // _NS
_SUB = 128
_NBUF = 4
_PAIRS_PER_W = _NPAIR // _NW
_IDXR_PER_W = _PAIRS_PER_W // _SUB

_mesh = plsc.VectorSubcoreMesh(
    core_axis_name="c", subcore_axis_name="s",
    num_cores=_NC, num_subcores=_NS,
)


def _pairidx_body(even_ref, odd_ref, out_ref):
    out_ref[...] = even_ref[...] * _VOCAB + odd_ref[...]


_pairidx = pl.pallas_call(
    _pairidx_body,
    out_shape=jax.ShapeDtypeStruct((_NPAIR // _SUB, _SUB), jnp.int32),
    grid=(8,),
    in_specs=[
        pl.BlockSpec((_NPAIR // _SUB // 8, _SUB), lambda i: (i, 0)),
        pl.BlockSpec((_NPAIR // _SUB // 8, _SUB), lambda i: (i, 0)),
    ],
    out_specs=pl.BlockSpec((_NPAIR // _SUB // 8, _SUB), lambda i: (i, 0)),
)


@functools.partial(
    pl.kernel,
    out_type=jax.ShapeDtypeStruct((_NPAIR, 2 * _D), jnp.float32),
    mesh=_mesh,
    scratch_types=[
        pltpu.VMEM((_VOCAB * _D,), jnp.float32),
        pltpu.VMEM((_IDXR_PER_W // 2, _SUB), jnp.int32),
        pltpu.VMEM((_NBUF * _SUB, 2 * _D), jnp.float32),
        pltpu.VMEM_SHARED((_PT_ROWS, 2 * _D), jnp.float32),
        pltpu.SemaphoreType.DMA,
        pltpu.SemaphoreType.DMA,
        pltpu.SemaphoreType.DMA,
        pltpu.SemaphoreType.DMA,
        pltpu.SemaphoreType.DMA,
        pltpu.SemaphoreType.DMA,
    ],
)
def _embed(pidx_hbm, wflat_hbm, out_hbm,
           wv, idxall, rows_v, pt_sh, isem, gsem, osem0, osem1, osem2, osem3):
    cid = lax.axis_index("c")
    sid = lax.axis_index("s")
    wid = sid * _NC + cid
    idxr0 = wid * _IDXR_PER_W
    pair0 = wid * _PAIRS_PER_W
    osems = (osem0, osem1, osem2, osem3)

    _HALF = _IDXR_PER_W // 2
    idesc = pltpu.async_copy(
        pidx_hbm.at[pl.ds(idxr0, _HALF)], idxall, isem)

    pltpu.sync_copy(wflat_hbm, wv)
    p0 = sid * _PT_PER_S

    def build_row(r, carry):
        a = (p0 + r) // _VOCAB
        b = (p0 + r) % _VOCAB
        for q in range(4):
            rows_v[r, pl.ds(q * 16, 16)] = wv[pl.ds(a * _D + q * 16, 16)]
            rows_v[r, pl.ds(_D + q * 16, 16)] = wv[pl.ds(b * _D + q * 16, 16)]
        return carry

    lax.fori_loop(0, _PT_PER_S, build_row, 0)
    pltpu.sync_copy(rows_v.at[pl.ds(0, _PT_PER_S)], pt_sh.at[pl.ds(p0, _PT_PER_S)])
    idesc.wait()
    plsc.subcore_barrier()

    def fire_gather(r, b):
        pltpu.async_copy(
            pt_sh.at[idxall.at[r]],
            rows_v.at[pl.ds(b * _SUB, _SUB)],
            gsem,
        )

    def fire_write(g, b):
        pltpu.async_copy(
            rows_v.at[pl.ds(b * _SUB, _SUB)],
            out_hbm.at[pl.ds(pair0 + g * _SUB, _SUB)],
            osems[b],
        )

    def drain_write(b):
        pltpu.make_async_copy(
            rows_v.at[pl.ds(b * _SUB, _SUB)],
            out_hbm.at[pl.ds(0, _SUB)],
            osems[b],
        ).wait()

    def drain_gather(r, b):
        pltpu.make_async_copy(
            pt_sh.at[idxall.at[r]],
            rows_v.at[pl.ds(b * _SUB, _SUB)],
            gsem,
        ).wait()

    for block in range(2):
        g0 = block * _HALF
        if block > 0:
            pltpu.sync_copy(
                pidx_hbm.at[pl.ds(idxr0 + g0, _HALF)], idxall)

        def body(h, carry, g0=g0):
            for b in range(_NBUF):
                r = _NBUF * h + b

                @pl.when(h >= 1)
                def _():
                    drain_write(b)

                fire_gather(r, b)
                pb = (b - 1) % _NBUF
                if b == 0:
                    @pl.when(h >= 1)
                    def _():
                        drain_gather(r - 1, pb)
                        fire_write(g0 + r - 1, pb)
                else:
                    drain_gather(r - 1, pb)
                    fire_write(g0 + r - 1, pb)
            return carry

        lax.fori_loop(0, _HALF // _NBUF, body, 0)
        last_b = _NBUF - 1
        drain_gather(_HALF - 1, last_b)
        fire_write(g0 + _HALF - 1, last_b)
        for b in range(_NBUF):
            drain_write(b)


_TBLK = 2048


def _transpose_body(in_ref, out_ref):
    blk = in_ref[...].reshape(_TBLK, 2 * _D)
    out_ref[...] = blk.T.reshape(2, _D, _TBLK)


_transpose = pl.pallas_call(
    _transpose_body,
    out_shape=jax.ShapeDtypeStruct((_COLS, _D, _ROWS), jnp.float32),
    grid=(_COLS // 2, _ROWS // _TBLK),
    in_specs=[
        pl.BlockSpec((1, _TBLK, 2 * _D), lambda p, c: (p, c, 0)),
    ],
    out_specs=pl.BlockSpec((2, _D, _TBLK), lambda p, c: (p, 0, c)),
)


def kernel(x, weight):
    x3 = x.reshape(_ROWS, _COLS // 2, 2).astype(jnp.int32)
    even = x3[:, :, 0].T.reshape(_NPAIR // _SUB, _SUB)
    odd = x3[:, :, 1].T.reshape(_NPAIR // _SUB, _SUB)
    pidx = _pairidx(even, odd)
    wflat = weight.reshape(_VOCAB * _D)
    pairs = _embed(pidx, wflat)
    out3 = _transpose(pairs.reshape(_COLS // 2, _ROWS, 2 * _D))
    return jnp.transpose(out3, (2, 0, 1))


# speedup vs baseline: 7.7441x; 1.3602x over previous
import functools

import jax
import jax.numpy as jnp
from jax import lax
from jax.experimental import pallas as pl
from jax.experimental.pallas import tpu as pltpu
from jax.experimental.pallas import tpu_sc as plsc

_VOCAB = 56
_D = 64
_ROWS, _COLS = 16384, 200
_B = _ROWS * _COLS
_NPAIR = _B // 2
_PT_ROWS = _VOCAB * _VOCAB
_NC, _NS = 2, 16
_NW = _NC * _NS
_PT_PER_S = _PT_ROWS // _NS
_SUB = 128
_NBUF = 4
_PAIRS_PER_W = _NPAIR // _NW
_IDXR_PER_W = _PAIRS_PER_W // _SUB

_mesh = plsc.VectorSubcoreMesh(
    core_axis_name="c", subcore_axis_name="s",
    num_cores=_NC, num_subcores=_NS,
)


def _pairidx_body(even_ref, odd_ref, out_ref):
    out_ref[...] = even_ref[...] * _VOCAB + odd_ref[...]


_pairidx = pl.pallas_call(
    _pairidx_body,
    out_shape=jax.ShapeDtypeStruct((_NPAIR // _SUB, _SUB), jnp.int32),
    grid=(8,),
    in_specs=[
        pl.BlockSpec((_NPAIR // _SUB // 8, _SUB), lambda i: (i, 0)),
        pl.BlockSpec((_NPAIR // _SUB // 8, _SUB), lambda i: (i, 0)),
    ],
    out_specs=pl.BlockSpec((_NPAIR // _SUB // 8, _SUB), lambda i: (i, 0)),
)


@functools.partial(
    pl.kernel,
    out_type=jax.ShapeDtypeStruct((_NPAIR, 2 * _D), jnp.float32),
    mesh=_mesh,
    scratch_types=[
        pltpu.VMEM((_VOCAB * _D,), jnp.float32),
        pltpu.VMEM((_IDXR_PER_W // 2, _SUB), jnp.int32),
        pltpu.VMEM((_NBUF * _SUB, 2 * _D), jnp.float32),
        pltpu.VMEM_SHARED((_PT_ROWS, 2 * _D), jnp.float32),
        pltpu.SemaphoreType.DMA,
        pltpu.SemaphoreType.DMA,
        pltpu.SemaphoreType.DMA,
        pltpu.SemaphoreType.DMA,
        pltpu.SemaphoreType.DMA,
        pltpu.SemaphoreType.DMA,
    ],
)
def _embed(pidx_hbm, wflat_hbm, out_hbm,
           wv, idxall, rows_v, pt_sh, isem, gsem, osem0, osem1, osem2, osem3):
    cid = lax.axis_index("c")
    sid = lax.axis_index("s")
    wid = sid * _NC + cid
    idxr0 = wid * _IDXR_PER_W
    pair0 = wid * _PAIRS_PER_W
    osems = (osem0, osem1, osem2, osem3)

    _HALF = _IDXR_PER_W // 2
    idesc = pltpu.async_copy(
        pidx_hbm.at[pl.ds(idxr0, _HALF)], idxall, isem)

    pltpu.sync_copy(wflat_hbm, wv)
    p0 = sid * _PT_PER_S

    def build_row(r, carry):
        a = (p0 + r) // _VOCAB
        b = (p0 + r) % _VOCAB
        for q in range(4):
            rows_v[r, pl.ds(q * 16, 16)] = wv[pl.ds(a * _D + q * 16, 16)]
            rows_v[r, pl.ds(_D + q * 16, 16)] = wv[pl.ds(b * _D + q * 16, 16)]
        return carry

    lax.fori_loop(0, _PT_PER_S, build_row, 0)
    pltpu.sync_copy(rows_v.at[pl.ds(0, _PT_PER_S)], pt_sh.at[pl.ds(p0, _PT_PER_S)])
    idesc.wait()
    plsc.subcore_barrier()

    def fire_gather(r, b):
        pltpu.async_copy(
            pt_sh.at[idxall.at[r]],
            rows_v.at[pl.ds(b * _SUB, _SUB)],
            gsem,
        )

    def fire_write(g, b):
        pltpu.async_copy(
            rows_v.at[pl.ds(b * _SUB, _SUB)],
            out_hbm.at[pl.ds(pair0 + g * _SUB, _SUB)],
            osems[b],
        )

    def drain_write(b):
        pltpu.make_async_copy(
            rows_v.at[pl.ds(b * _SUB, _SUB)],
            out_hbm.at[pl.ds(0, _SUB)],
            osems[b],
        ).wait()

    def drain_gather(r, b):
        pltpu.make_async_copy(
            pt_sh.at[idxall.at[r]],
            rows_v.at[pl.ds(b * _SUB, _SUB)],
            gsem,
        ).wait()

    for block in range(2):
        g0 = block * _HALF
        if block > 0:
            pltpu.sync_copy(
                pidx_hbm.at[pl.ds(idxr0 + g0, _HALF)], idxall)

        def body(h, carry, g0=g0):
            for b in range(_NBUF):
                r = _NBUF * h + b

                @pl.when(h >= 1)
                def _():
                    drain_write(b)

                fire_gather(r, b)
                pb = (b - 1) % _NBUF
                if b == 0:
                    @pl.when(h >= 1)
                    def _():
                        drain_gather(r - 1, pb)
                        fire_write(g0 + r - 1, pb)
                else:
                    drain_gather(r - 1, pb)
                    fire_write(g0 + r - 1, pb)
            return carry

        lax.fori_loop(0, _HALF // _NBUF, body, 0)
        last_b = _NBUF - 1
        drain_gather(_HALF - 1, last_b)
        fire_write(g0 + _HALF - 1, last_b)
        for b in range(_NBUF):
            drain_write(b)


_TBLK = 8192


def _transpose_body(in_ref, out_ref):
    blk = in_ref[...].reshape(_TBLK, 2 * _D)
    out_ref[...] = blk.T.reshape(2, _D, _TBLK)


_transpose = pl.pallas_call(
    _transpose_body,
    out_shape=jax.ShapeDtypeStruct((_COLS, _D, _ROWS), jnp.float32),
    grid=(_COLS // 2, _ROWS // _TBLK),
    in_specs=[
        pl.BlockSpec((1, _TBLK, 2 * _D), lambda p, c: (p, c, 0)),
    ],
    out_specs=pl.BlockSpec((2, _D, _TBLK), lambda p, c: (p, 0, c)),
)


def kernel(x, weight):
    x3 = x.reshape(_ROWS, _COLS // 2, 2).astype(jnp.int32)
    even = x3[:, :, 0].T.reshape(_NPAIR // _SUB, _SUB)
    odd = x3[:, :, 1].T.reshape(_NPAIR // _SUB, _SUB)
    pidx = _pairidx(even, odd)
    wflat = weight.reshape(_VOCAB * _D)
    pairs = _embed(pidx, wflat)
    out3 = _transpose(pairs.reshape(_COLS // 2, _ROWS, 2 * _D))
    return jnp.transpose(out3, (2, 0, 1))


# speedup vs baseline: 7.8518x; 1.0139x over previous
import functools

import jax
import jax.numpy as jnp
from jax import lax
from jax.experimental import pallas as pl
from jax.experimental.pallas import tpu as pltpu
from jax.experimental.pallas import tpu_sc as plsc

_VOCAB = 56
_D = 64
_ROWS, _COLS = 16384, 200
_B = _ROWS * _COLS
_NPAIR = _B // 2
_PT_ROWS = _VOCAB * _VOCAB
_NC, _NS = 2, 16
_NW = _NC * _NS
_PT_PER_S = _PT_ROWS // _NS
_SUB = 128
_NBUF = 4
_PAIRS_PER_W = _NPAIR // _NW
_IDXR_PER_W = _PAIRS_PER_W // _SUB

_mesh = plsc.VectorSubcoreMesh(
    core_axis_name="c", subcore_axis_name="s",
    num_cores=_NC, num_subcores=_NS,
)


def _pairidx_body(even_ref, odd_ref, out_ref):
    out_ref[...] = even_ref[...] * _VOCAB + odd_ref[...]


_pairidx = pl.pallas_call(
    _pairidx_body,
    out_shape=jax.ShapeDtypeStruct((_NPAIR // _SUB, _SUB), jnp.int32),
    grid=(8,),
    in_specs=[
        pl.BlockSpec((_NPAIR // _SUB // 8, _SUB), lambda i: (i, 0)),
        pl.BlockSpec((_NPAIR // _SUB // 8, _SUB), lambda i: (i, 0)),
    ],
    out_specs=pl.BlockSpec((_NPAIR // _SUB // 8, _SUB), lambda i: (i, 0)),
)


@functools.partial(
    pl.kernel,
    out_type=jax.ShapeDtypeStruct((_NPAIR, 2 * _D), jnp.float32),
    mesh=_mesh,
    scratch_types=[
        pltpu.VMEM((_VOCAB * _D,), jnp.float32),
        pltpu.VMEM((_IDXR_PER_W // 2, _SUB), jnp.int32),
        pltpu.VMEM((_NBUF * _SUB, 2 * _D), jnp.float32),
        pltpu.VMEM_SHARED((_PT_ROWS, 2 * _D), jnp.float32),
        pltpu.SemaphoreType.DMA,
        pltpu.SemaphoreType.DMA,
        pltpu.SemaphoreType.DMA,
        pltpu.SemaphoreType.DMA,
        pltpu.SemaphoreType.DMA,
        pltpu.SemaphoreType.DMA,
    ],
)
def _embed(pidx_hbm, wflat_hbm, out_hbm,
           wv, idxall, rows_v, pt_sh, isem, gsem, osem0, osem1, osem2, osem3):
    cid = lax.axis_index("c")
    sid = lax.axis_index("s")
    wid = sid * _NC + cid
    idxr0 = wid * _IDXR_PER_W
    pair0 = wid * _PAIRS_PER_W
    osems = (osem0, osem1, osem2, osem3)

    _HALF = _IDXR_PER_W // 2
    idesc = pltpu.async_copy(
        pidx_hbm.at[pl.ds(idxr0, _HALF)], idxall, isem)

    pltpu.sync_copy(wflat_hbm, wv)
    p0 = sid * _PT_PER_S

    def build_row(r, carry):
        a = (p0 + r) // _VOCAB
        b = (p0 + r) % _VOCAB
        for q in range(4):
            rows_v[r, pl.ds(q * 16, 16)] = wv[pl.ds(a * _D + q * 16, 16)]
            rows_v[r, pl.ds(_D + q * 16, 16)] = wv[pl.ds(b * _D + q * 16, 16)]
        return carry

    lax.fori_loop(0, _PT_PER_S, build_row, 0)
    pltpu.sync_copy(rows_v.at[pl.ds(0, _PT_PER_S)], pt_sh.at[pl.ds(p0, _PT_PER_S)])
    idesc.wait()
    plsc.subcore_barrier()

    def fire_gather(r, b):
        pltpu.async_copy(
            pt_sh.at[idxall.at[r]],
            rows_v.at[pl.ds(b * _SUB, _SUB)],
            gsem,
        )

    def fire_write(g, b):
        pltpu.async_copy(
            rows_v.at[pl.ds(b * _SUB, _SUB)],
            out_hbm.at[pl.ds(pair0 + g * _SUB, _SUB)],
            osems[b],
        )

    def drain_write(b):
        pltpu.make_async_copy(
            rows_v.at[pl.ds(b * _SUB, _SUB)],
            out_hbm.at[pl.ds(0, _SUB)],
            osems[b],
        ).wait()

    def drain_gather(r, b):
        pltpu.make_async_copy(
            pt_sh.at[idxall.at[r]],
            rows_v.at[pl.ds(b * _SUB, _SUB)],
            gsem,
        ).wait()

    for block in range(2):
        g0 = block * _HALF
        if block > 0:
            pltpu.sync_copy(
                pidx_hbm.at[pl.ds(idxr0 + g0, _HALF)], idxall)

        def body(h, carry, g0=g0):
            for b in range(_NBUF):
                r = _NBUF * h + b

                @pl.when(h >= 1)
                def _():
                    drain_write(b)

                fire_gather(r, b)
                pb = (b - 1) % _NBUF
                if b == 0:
                    @pl.when(h >= 1)
                    def _():
                        drain_gather(r - 1, pb)
                        fire_write(g0 + r - 1, pb)
                else:
                    drain_gather(r - 1, pb)
                    fire_write(g0 + r - 1, pb)
            return carry

        lax.fori_loop(0, _HALF // _NBUF, body, 0)
        last_b = _NBUF - 1
        drain_gather(_HALF - 1, last_b)
        fire_write(g0 + _HALF - 1, last_b)
        for b in range(_NBUF):
            drain_write(b)


_TBLK = 16384


def _transpose_body(in_ref, out_ref):
    blk = in_ref[...].reshape(_TBLK, 2 * _D)
    out_ref[...] = blk.T.reshape(2, _D, _TBLK)


_transpose = pl.pallas_call(
    _transpose_body,
    out_shape=jax.ShapeDtypeStruct((_COLS, _D, _ROWS), jnp.float32),
    grid=(_COLS // 2, _ROWS // _TBLK),
    in_specs=[
        pl.BlockSpec((1, _TBLK, 2 * _D), lambda p, c: (p, c, 0)),
    ],
    out_specs=pl.BlockSpec((2, _D, _TBLK), lambda p, c: (p, 0, c)),
)


def kernel(x, weight):
    x3 = x.reshape(_ROWS, _COLS // 2, 2).astype(jnp.int32)
    even = x3[:, :, 0].T.reshape(_NPAIR // _SUB, _SUB)
    odd = x3[:, :, 1].T.reshape(_NPAIR // _SUB, _SUB)
    pidx = _pairidx(even, odd)
    wflat = weight.reshape(_VOCAB * _D)
    pairs = _embed(pidx, wflat)
    out3 = _transpose(pairs.reshape(_COLS // 2, _ROWS, 2 * _D))
    return jnp.transpose(out3, (2, 0, 1))
